# initial kernel scaffold (unmeasured)
import jax
import jax.numpy as jnp
from jax import lax
from jax.experimental import pallas as pl
from jax.experimental.pallas import tpu as pltpu


def kernel(
    x,
):
    def body(*refs):
        pass

    out_shape = jax.ShapeDtypeStruct(..., jnp.float32)
    return pl.pallas_call(body, out_shape=out_shape)(...)


# baseline (device time: 17804 ns/iter reference)
import functools

import jax
import jax.numpy as jnp
from jax import lax
from jax.experimental import pallas as pl
from jax.experimental.pallas import tpu as pltpu


def kernel(x):
    m, n = x.shape

    def body(x_ref, out_ref, row_recv, col_recv, row_sems, col_sems):
        my_x = lax.axis_index("x")
        my_y = lax.axis_index("y")
        nbr_x = (1 - my_x, my_y)
        nbr_y = (my_x, 1 - my_y)

        barrier_sem = pltpu.get_barrier_semaphore()
        pl.semaphore_signal(barrier_sem, inc=1, device_id=nbr_x,
                            device_id_type=pl.DeviceIdType.MESH)
        pl.semaphore_signal(barrier_sem, inc=1, device_id=nbr_y,
                            device_id_type=pl.DeviceIdType.MESH)
        pl.semaphore_wait(barrier_sem, 2)

        src_row = pl.multiple_of(jnp.where(my_x == 0, m - 8, 0), 8)
        src_col = pl.multiple_of(jnp.where(my_y == 0, n - 128, 0), 128)

        row_rdma = pltpu.make_async_remote_copy(
            src_ref=x_ref.at[pl.ds(src_row, 8), :],
            dst_ref=row_recv,
            send_sem=row_sems.at[0],
            recv_sem=row_sems.at[1],
            device_id=nbr_x,
            device_id_type=pl.DeviceIdType.MESH,
        )
        col_rdma = pltpu.make_async_remote_copy(
            src_ref=x_ref.at[:, pl.ds(src_col, 128)],
            dst_ref=col_recv,
            send_sem=col_sems.at[0],
            recv_sem=col_sems.at[1],
            device_id=nbr_y,
            device_id_type=pl.DeviceIdType.MESH,
        )
        row_rdma.start()
        col_rdma.start()
        row_rdma.wait()
        col_rdma.wait()

        xv = x_ref[:, :]
        rrow = jnp.where(my_x == 1, row_recv[7:8, :], row_recv[0:1, :])
        rcol = jnp.where(my_y == 1, col_recv[:, 127:128], col_recv[:, 0:1])
        zrow = jnp.zeros_like(rrow)
        zcol = jnp.zeros_like(rcol)
        north = jnp.where(my_x == 1, rrow, zrow)
        south = jnp.where(my_x == 0, rrow, zrow)
        west = jnp.where(my_y == 1, rcol, zcol)
        east = jnp.where(my_y == 0, rcol, zcol)

        nv = jnp.concatenate([north, xv[:-1, :]], axis=0)
        sv = jnp.concatenate([xv[1:, :], south], axis=0)
        wv = jnp.concatenate([west, xv[:, :-1]], axis=1)
        ev = jnp.concatenate([xv[:, 1:], east], axis=1)
        out = 0.5 * xv + 0.125 * (nv + sv + wv + ev)

        ri = lax.broadcasted_iota(jnp.int32, (m, n), 0)
        ci = lax.broadcasted_iota(jnp.int32, (m, n), 1)
        boundary = (
            ((my_x == 0) & (ri == 0))
            | ((my_x == 1) & (ri == m - 1))
            | ((my_y == 0) & (ci == 0))
            | ((my_y == 1) & (ci == n - 1))
        )
        out_ref[:, :] = jnp.where(boundary, xv, out)

        @functools.partial(pl.run_scoped, sem2=pltpu.SemaphoreType.REGULAR)
        def _(sem2):
            pl.semaphore_signal(sem2, inc=1, device_id=nbr_x,
                                device_id_type=pl.DeviceIdType.MESH)
            pl.semaphore_signal(sem2, inc=1, device_id=nbr_y,
                                device_id_type=pl.DeviceIdType.MESH)
            pl.semaphore_wait(sem2, 2)

    return pl.pallas_call(
        body,
        out_shape=jax.ShapeDtypeStruct((m, n), x.dtype),
        in_specs=[pl.BlockSpec(memory_space=pltpu.VMEM)],
        out_specs=pl.BlockSpec(memory_space=pltpu.VMEM),
        scratch_shapes=[
            pltpu.VMEM((8, n), x.dtype),
            pltpu.VMEM((m, 128), x.dtype),
            pltpu.SemaphoreType.DMA((2,)),
            pltpu.SemaphoreType.DMA((2,)),
        ],
        compiler_params=pltpu.CompilerParams(collective_id=0),
    )(x)


# device time: 15239 ns/iter; 1.1683x vs baseline; 1.1683x over previous
import functools

import jax
import jax.numpy as jnp
from jax import lax
from jax.experimental import pallas as pl
from jax.experimental.pallas import tpu as pltpu


def kernel(x):
    m, n = x.shape

    def body(x_ref, out_ref, row_recv, col_recv, row_sems, col_sems):
        my_x = lax.axis_index("x")
        my_y = lax.axis_index("y")
        nbr_x = (1 - my_x, my_y)
        nbr_y = (my_x, 1 - my_y)

        barrier_sem = pltpu.get_barrier_semaphore()
        pl.semaphore_signal(barrier_sem, inc=1, device_id=nbr_x,
                            device_id_type=pl.DeviceIdType.MESH)
        pl.semaphore_signal(barrier_sem, inc=1, device_id=nbr_y,
                            device_id_type=pl.DeviceIdType.MESH)
        pl.semaphore_wait(barrier_sem, 2)

        src_row = pl.multiple_of(jnp.where(my_x == 0, m - 8, 0), 8)
        src_col = pl.multiple_of(jnp.where(my_y == 0, n - 128, 0), 128)

        row_rdma = pltpu.make_async_remote_copy(
            src_ref=x_ref.at[pl.ds(src_row, 8), :],
            dst_ref=row_recv,
            send_sem=row_sems.at[0],
            recv_sem=row_sems.at[1],
            device_id=nbr_x,
            device_id_type=pl.DeviceIdType.MESH,
        )
        col_rdma = pltpu.make_async_remote_copy(
            src_ref=x_ref.at[:, pl.ds(src_col, 128)],
            dst_ref=col_recv,
            send_sem=col_sems.at[0],
            recv_sem=col_sems.at[1],
            device_id=nbr_y,
            device_id_type=pl.DeviceIdType.MESH,
        )
        row_rdma.start()
        col_rdma.start()

        xv = x_ref[:, :]
        zrow = jnp.zeros((1, n), xv.dtype)
        zcol = jnp.zeros((m, 1), xv.dtype)
        nv = jnp.concatenate([zrow, xv[:-1, :]], axis=0)
        sv = jnp.concatenate([xv[1:, :], zrow], axis=0)
        wv = jnp.concatenate([zcol, xv[:, :-1]], axis=1)
        ev = jnp.concatenate([xv[:, 1:], zcol], axis=1)
        out = 0.5 * xv + 0.125 * (nv + sv + wv + ev)

        ri = lax.broadcasted_iota(jnp.int32, (m, n), 0)
        ci = lax.broadcasted_iota(jnp.int32, (m, n), 1)
        boundary = (
            ((my_x == 0) & (ri == 0))
            | ((my_x == 1) & (ri == m - 1))
            | ((my_y == 0) & (ci == 0))
            | ((my_y == 1) & (ci == n - 1))
        )
        out_ref[:, :] = jnp.where(boundary, xv, out)

        row_rdma.wait()
        rrow = jnp.where(my_x == 1, row_recv[7:8, :], row_recv[0:1, :])
        rb = pl.multiple_of(jnp.where(my_x == 1, 0, m - 8), 8)
        tgt_sub = jnp.where(my_x == 1, 0, 7)
        blk = out_ref[pl.ds(rb, 8), :]
        sub_i = lax.broadcasted_iota(jnp.int32, (8, n), 0)
        lane_i = lax.broadcasted_iota(jnp.int32, (8, n), 1)
        col_bdry = ((my_y == 0) & (lane_i == 0)) | ((my_y == 1) & (lane_i == n - 1))
        add = jnp.where((sub_i == tgt_sub) & ~col_bdry,
                        0.125 * jnp.broadcast_to(rrow, (8, n)), 0.0)
        out_ref[pl.ds(rb, 8), :] = blk + add

        col_rdma.wait()
        rcol = jnp.where(my_y == 1, col_recv[:, 127:128], col_recv[:, 0:1])
        cb = pl.multiple_of(jnp.where(my_y == 1, 0, n - 128), 128)
        tgt_lane = jnp.where(my_y == 1, 0, 127)
        cblk = out_ref[:, pl.ds(cb, 128)]
        lane_c = lax.broadcasted_iota(jnp.int32, (m, 128), 1)
        sub_c = lax.broadcasted_iota(jnp.int32, (m, 128), 0)
        row_bdry = ((my_x == 0) & (sub_c == 0)) | ((my_x == 1) & (sub_c == m - 1))
        cadd = jnp.where((lane_c == tgt_lane) & ~row_bdry,
                         0.125 * jnp.broadcast_to(rcol, (m, 128)), 0.0)
        out_ref[:, pl.ds(cb, 128)] = cblk + cadd

        @functools.partial(pl.run_scoped, sem2=pltpu.SemaphoreType.REGULAR)
        def _(sem2):
            pl.semaphore_signal(sem2, inc=1, device_id=nbr_x,
                                device_id_type=pl.DeviceIdType.MESH)
            pl.semaphore_signal(sem2, inc=1, device_id=nbr_y,
                                device_id_type=pl.DeviceIdType.MESH)
            pl.semaphore_wait(sem2, 2)

    return pl.pallas_call(
        body,
        out_shape=jax.ShapeDtypeStruct((m, n), x.dtype),
        in_specs=[pl.BlockSpec(memory_space=pltpu.VMEM)],
        out_specs=pl.BlockSpec(memory_space=pltpu.VMEM),
        scratch_shapes=[
            pltpu.VMEM((8, n), x.dtype),
            pltpu.VMEM((m, 128), x.dtype),
            pltpu.SemaphoreType.DMA((2,)),
            pltpu.SemaphoreType.DMA((2,)),
        ],
        compiler_params=pltpu.CompilerParams(collective_id=0),
    )(x)
